# baseline (speedup 1.0000x reference)
import functools

import jax
import jax.numpy as jnp
import numpy as np
from jax import lax
from jax.experimental import pallas as pl
from jax.experimental.pallas import tpu as pltpu
from jax.experimental.pallas import tpu_sc as plsc

_VOCABS = [9, 16, 7, 15, 6, 5, 2, 40]
_DIMS = [3, 5, 2, 5, 2, 2, 2, 13]
_OFFS = [0, 3, 8, 10, 15, 17, 19, 21]
_VBASE = [0, 9, 25, 32, 47, 53, 58, 60]
_B = 4096
_H = 50
_GROUPS = [(1, 3), (7, 6), (0, 2), (4, 5)]
_GROUP_BASE = [0, 240, 320, 383]
_NROWS = 416

_NW = 32
_BPW = _B // _NW


def _combination_matrix() -> np.ndarray:
    s = np.zeros((_NROWS, 100), np.float32)
    for gi, (ta, tb) in enumerate(_GROUPS):
        va, vb = _VOCABS[ta], _VOCABS[tb]
        for a in range(va):
            for b in range(vb):
                r = _GROUP_BASE[gi] + a * vb + b
                s[r, _VBASE[ta] + a] = 1.0
                s[r, _VBASE[tb] + b] = 1.0
    return s


def _tc_build_body(e0, e1, e2, e3, e4, e5, e6, e7, x_ref, w1_ref, b1_ref,
                   w2_ref, b2_ref, s_ref, c_ref, nt_ref, w2bc_ref):
    embs = [e0, e1, e2, e3, e4, e5, e6, e7]
    w1 = w1_ref[...]
    fused = [
        jnp.dot(embs[i][...], w1[_OFFS[i]:_OFFS[i] + _DIMS[i], :],
                preferred_element_type=jnp.float32)
        for i in range(8)
    ]
    f = jnp.concatenate(fused, axis=0)
    c_ref[...] = jnp.dot(s_ref[...], f, preferred_element_type=jnp.float32)
    xnum = x_ref[...][:, 8:14]
    n = lax.dot_general(w1[34:40, :], xnum, (((0,), (1,)), ((), ())),
                        preferred_element_type=jnp.float32)
    n = n + b1_ref[...]
    for w in range(_NW):
        nt_ref[w] = n[:, w * _BPW:(w + 1) * _BPW]
    w2b2 = jnp.concatenate([w2_ref[...], b2_ref[...]], axis=0)
    w2bc_ref[...] = jnp.dot(w2b2, jnp.ones((1, 16), jnp.float32),
                            preferred_element_type=jnp.float32)


_SC_MESH = plsc.VectorSubcoreMesh(core_axis_name="c", subcore_axis_name="s",
                                  num_cores=2, num_subcores=16)


@functools.partial(
    pl.kernel,
    out_type=jax.ShapeDtypeStruct((_B,), jnp.float32),
    mesh=_SC_MESH,
    compiler_params=pltpu.CompilerParams(needs_layout_passes=False),
    scratch_types=[
        pltpu.VMEM((_NROWS * _H,), jnp.float32),
        pltpu.VMEM((_H * _BPW,), jnp.float32),
        pltpu.VMEM((51 * 16,), jnp.float32),
        pltpu.VMEM((_BPW * 14,), jnp.float32),
        pltpu.VMEM((_BPW,), jnp.float32),
    ],
)
def _sc_forward(c_hbm, nt_hbm, w2bc_hbm, x_hbm, out_hbm, c_v, n_v, w2_v, x_v,
                o_v):
    wid = lax.axis_index("s") * 2 + lax.axis_index("c")
    base = wid * _BPW
    pltpu.sync_copy(c_hbm, c_v)
    pltpu.sync_copy(nt_hbm.at[pl.ds(wid * (_H * _BPW), _H * _BPW)], n_v)
    pltpu.sync_copy(w2bc_hbm, w2_v)
    pltpu.sync_copy(x_hbm.at[pl.ds(base * 14, _BPW * 14)], x_v)

    lane = lax.iota(jnp.int32, 16)

    def per_group(g, carry):
        rowbase = (lane + g * 16) * 14

        def col(t):
            return plsc.load_gather(x_v, [rowbase + t]).astype(jnp.int32)

        ci = [col(t) for t in range(8)]
        fb = []
        for gi, (ta, tb) in enumerate(_GROUPS):
            r = ci[ta] * _VOCABS[tb] + ci[tb] + _GROUP_BASE[gi]
            fb.append(r * _H)
        out_acc = jnp.zeros((16,), jnp.float32)
        for j in range(_H):
            acc = plsc.load_gather(c_v, [fb[0] + j])
            acc = acc + plsc.load_gather(c_v, [fb[1] + j])
            acc = acc + plsc.load_gather(c_v, [fb[2] + j])
            acc = acc + plsc.load_gather(c_v, [fb[3] + j])
            acc = acc + n_v[pl.ds(j * _BPW + g * 16, 16)]
            acc = jnp.maximum(acc, 0.0)
            out_acc = out_acc + acc * w2_v[pl.ds(j * 16, 16)]
        z = out_acc + w2_v[pl.ds(_H * 16, 16)]
        sig = 1.0 / (1.0 + jnp.exp(-z))
        o_v[pl.ds(g * 16, 16)] = sig
        return carry

    lax.fori_loop(0, _BPW // 16, per_group, None)
    pltpu.sync_copy(o_v, out_hbm.at[pl.ds(base, _BPW)])


def kernel(x, emb0, emb1, emb2, emb3, emb4, emb5, emb6, emb7, W1, b1, W2, b2):
    s_const = jnp.asarray(_combination_matrix())
    c, nt, w2bc = pl.pallas_call(
        _tc_build_body,
        out_shape=(jax.ShapeDtypeStruct((_NROWS, _H), jnp.float32),
                   jax.ShapeDtypeStruct((_NW, _H, _BPW), jnp.float32),
                   jax.ShapeDtypeStruct((51, 16), jnp.float32)),
    )(emb0, emb1, emb2, emb3, emb4, emb5, emb6, emb7, x,
      W1, b1.reshape(_H, 1), W2, b2.reshape(1, 1), s_const)
    out = _sc_forward(c.reshape(_NROWS * _H), nt.reshape(_NW * _H * _BPW),
                      w2bc.reshape(51 * 16), x.reshape(_B * 14))
    return out.reshape(_B, 1)


# speedup vs baseline: 1.0621x; 1.0621x over previous
import functools

import jax
import jax.numpy as jnp
from jax import lax
from jax.experimental import pallas as pl
from jax.experimental.pallas import tpu as pltpu
from jax.experimental.pallas import tpu_sc as plsc

_VOCABS = [9, 16, 7, 15, 6, 5, 2, 40]
_DIMS = [3, 5, 2, 5, 2, 2, 2, 13]
_OFFS = [0, 3, 8, 10, 15, 17, 19, 21]
_VBASE = [0, 9, 25, 32, 47, 53, 58, 60]
_B = 4096
_H = 50
_CROWS = 104

_NW = 32
_BPW = _B // _NW


def _tc_build_body(e_ref, wall_ref, x_ref, c_ref, nt_ref):
    wall = wall_ref[...]
    w1 = wall[0:40, :]
    w2r = wall[41:42, :]
    b2r = wall[42:43, :]
    e = e_ref[...]
    fused = [
        jnp.dot(e[_VBASE[i]:_VBASE[i] + _VOCABS[i], 0:_DIMS[i]],
                w1[_OFFS[i]:_OFFS[i] + _DIMS[i], :],
                preferred_element_type=jnp.float32)
        for i in range(8)
    ]
    c_ref[...] = jnp.concatenate(
        fused + [w2r, b2r, jnp.zeros((2, _H), jnp.float32)], axis=0)
    xnum1 = jnp.concatenate(
        [x_ref[...][:, 8:14], jnp.ones((_B, 1), jnp.float32)], axis=1)
    n = lax.dot_general(wall[34:41, :], xnum1, (((0,), (1,)), ((), ())),
                        preferred_element_type=jnp.float32)
    for w in range(_NW):
        nt_ref[w] = n[:, w * _BPW:(w + 1) * _BPW]


_SC_MESH = plsc.VectorSubcoreMesh(core_axis_name="c", subcore_axis_name="s",
                                  num_cores=2, num_subcores=16)


@functools.partial(
    pl.kernel,
    out_type=jax.ShapeDtypeStruct((_B,), jnp.float32),
    mesh=_SC_MESH,
    compiler_params=pltpu.CompilerParams(needs_layout_passes=False),
    scratch_types=[
        pltpu.VMEM((_CROWS * _H,), jnp.float32),
        pltpu.VMEM((_H * _BPW,), jnp.float32),
        pltpu.VMEM((_BPW * 14,), jnp.float32),
        pltpu.VMEM((_BPW,), jnp.float32),
    ],
)
def _sc_forward(c_hbm, nt_hbm, x_hbm, out_hbm, c_v, n_v, x_v, o_v):
    wid = lax.axis_index("s") * 2 + lax.axis_index("c")
    base = wid * _BPW
    pltpu.sync_copy(c_hbm, c_v)
    pltpu.sync_copy(nt_hbm.at[pl.ds(wid * (_H * _BPW), _H * _BPW)], n_v)
    pltpu.sync_copy(x_hbm.at[pl.ds(base * 14, _BPW * 14)], x_v)

    lane = lax.iota(jnp.int32, 16)

    def per_group(g, carry):
        rowbase = (lane + g * 16) * 14

        def col(t):
            return plsc.load_gather(x_v, [rowbase + t]).astype(jnp.int32)

        fb = [(col(t) + _VBASE[t]) * _H for t in range(8)]
        out_acc = jnp.zeros((16,), jnp.float32)
        for j in range(_H):
            acc = plsc.load_gather(c_v, [fb[0] + j])
            for t in range(1, 8):
                acc = acc + plsc.load_gather(c_v, [fb[t] + j])
            acc = acc + n_v[pl.ds(j * _BPW + g * 16, 16)]
            acc = jnp.maximum(acc, 0.0)
            w2j = plsc.load_gather(c_v, [jnp.full((16,), 100 * _H + j,
                                                  jnp.int32)])
            out_acc = out_acc + acc * w2j
        b2 = plsc.load_gather(c_v, [jnp.full((16,), 101 * _H, jnp.int32)])
        z = out_acc + b2
        sig = 1.0 / (1.0 + jnp.exp(-z))
        o_v[pl.ds(g * 16, 16)] = sig
        return carry

    lax.fori_loop(0, _BPW // 16, per_group, None)
    pltpu.sync_copy(o_v, out_hbm.at[pl.ds(base, _BPW)])


def kernel(x, emb0, emb1, emb2, emb3, emb4, emb5, emb6, emb7, W1, b1, W2, b2):
    embs = [emb0, emb1, emb2, emb3, emb4, emb5, emb6, emb7]
    e = jnp.zeros((100, 16), jnp.float32)
    for i in range(8):
        e = e.at[_VBASE[i]:_VBASE[i] + _VOCABS[i], 0:_DIMS[i]].set(embs[i])
    wall = jnp.concatenate(
        [W1, b1.reshape(1, _H), W2.reshape(1, _H),
         jnp.pad(b2.reshape(1, 1), ((0, 0), (0, _H - 1)))], axis=0)
    c, nt = pl.pallas_call(
        _tc_build_body,
        out_shape=(jax.ShapeDtypeStruct((_CROWS, _H), jnp.float32),
                   jax.ShapeDtypeStruct((_NW, _H, _BPW), jnp.float32)),
    )(e, wall, x)
    out = _sc_forward(c.reshape(_CROWS * _H), nt.reshape(_NW * _H * _BPW),
                      x.reshape(_B * 14))
    return out.reshape(_B, 1)
